# Initial kernel scaffold; baseline (speedup 1.0000x reference)
import jax
import jax.numpy as jnp
from jax.experimental import pallas as pl


def kernel(tokens, W_gate, W1, b1, W2, b2):
    raise NotImplementedError("write your pallas kernel here")


# speedup vs baseline: 1.0676x; 1.0676x over previous
import functools

import jax
import jax.numpy as jnp
from jax import lax
from jax.experimental import pallas as pl
from jax.experimental.pallas import tpu as pltpu
from jax.experimental.pallas import tpu_sc as plsc

N = 2048
H = 768
E = 8
CAP = 1024
T = 256
GMAX = 23
GPAD = 32
XS_ROWS = GMAX * T
TRASH = XS_ROWS
XS_TOT = XS_ROWS + 8

NC, NS = 2, 16
NW = NC * NS
TOK_PER = N // NW
LANES = 16


def _routing_body(tok_ref, wg_ref, d1_ref, d2_ref, wa_ref, wb_ref, be_ref):
    x = tok_ref[...]
    wg = wg_ref[...]
    logits = lax.dot_general(x, wg, (((1,), (1,)), ((), ())),
                             preferred_element_type=jnp.float32)
    m = jnp.max(logits, axis=1, keepdims=True)
    ex = jnp.exp(logits - m)
    probs = ex / jnp.sum(ex, axis=1, keepdims=True)

    lane = lax.broadcasted_iota(jnp.int32, (N, E), 1)
    m1 = jnp.max(probs, axis=1, keepdims=True)
    i1 = jnp.min(jnp.where(probs == m1, lane, E), axis=1, keepdims=True)
    probs2 = jnp.where(lane == i1, -1.0, probs)
    m2 = jnp.max(probs2, axis=1, keepdims=True)
    i2 = jnp.min(jnp.where(probs2 == m2, lane, E), axis=1, keepdims=True)

    oh1 = (lane == i1).astype(jnp.float32)
    oh2 = (lane == i2).astype(jnp.float32)
    s = oh1 + oh2

    kc = 256
    row_i = lax.broadcasted_iota(jnp.int32, (N, kc), 0)
    col_i = lax.broadcasted_iota(jnp.int32, (N, kc), 1)
    c = jnp.zeros((N, E), jnp.float32)
    for k in range(N // kc):
        mask = (col_i + k * kc < row_i).astype(jnp.float32)
        c = c + lax.dot_general(mask, s[k * kc:(k + 1) * kc, :],
                                (((1,), (0,)), ((), ())),
                                preferred_element_type=jnp.float32)

    counts = jnp.sum(s, axis=0, keepdims=True)
    counts_i = counts.astype(jnp.int32)
    kept = counts_i <= CAP
    nblk = jnp.where(kept, (counts_i + (T - 1)) // T, 0)

    e_r = lax.broadcasted_iota(jnp.int32, (E, E), 0)
    e_c = lax.broadcasted_iota(jnp.int32, (E, E), 1)
    mex = (e_r < e_c).astype(jnp.float32)
    nblk_f = nblk.astype(jnp.float32)
    blkoff = lax.dot_general(nblk_f, mex, (((1,), (0,)), ((), ())),
                             preferred_element_type=jnp.float32)
    rowoff = blkoff * float(T)
    cum = blkoff + nblk_f

    r1 = jnp.sum(c * oh1, axis=1, keepdims=True)
    r2 = jnp.sum(c * oh2, axis=1, keepdims=True)
    ro1 = jnp.sum(rowoff * oh1, axis=1, keepdims=True)
    ro2 = jnp.sum(rowoff * oh2, axis=1, keepdims=True)
    keptf = kept.astype(jnp.float32)
    k1 = jnp.sum(keptf * oh1, axis=1, keepdims=True)
    k2 = jnp.sum(keptf * oh2, axis=1, keepdims=True)

    d1_ref[...] = jnp.where(k1 > 0., ro1 + r1, float(TRASH)).astype(jnp.int32)
    d2_ref[...] = jnp.where(k2 > 0., ro2 + r2, float(TRASH)).astype(jnp.int32)
    wa_ref[...] = jnp.broadcast_to(m1 * k1, (N, LANES))
    wb_ref[...] = jnp.broadcast_to(m2 * k2, (N, LANES))

    g_i = lax.broadcasted_iota(jnp.int32, (GPAD, E), 0).astype(jnp.float32)
    be = jnp.sum((cum <= g_i).astype(jnp.float32), axis=1, keepdims=True)
    be_ref[...] = be.astype(jnp.int32)


def _routing(flat_tokens, w_gate):
    return pl.pallas_call(
        _routing_body,
        out_shape=[
            jax.ShapeDtypeStruct((N, 1), jnp.int32),
            jax.ShapeDtypeStruct((N, 1), jnp.int32),
            jax.ShapeDtypeStruct((N, LANES), jnp.float32),
            jax.ShapeDtypeStruct((N, LANES), jnp.float32),
            jax.ShapeDtypeStruct((GPAD, 1), jnp.int32),
        ],
    )(flat_tokens, w_gate)


def _dispatch_body(tok_hbm, d1_hbm, d2_hbm, xs_hbm, d1_v, d2_v, rows_v, sem):
    c = lax.axis_index("c")
    s = lax.axis_index("s")
    wid = s * NC + c
    base = wid * TOK_PER
    pltpu.sync_copy(d1_hbm.at[pl.ds(base, TOK_PER)], d1_v)
    pltpu.sync_copy(d2_hbm.at[pl.ds(base, TOK_PER)], d2_v)
    pltpu.sync_copy(tok_hbm.at[pl.ds(base, TOK_PER)], rows_v)
    pltpu.async_copy(rows_v, xs_hbm.at[d1_v], sem).wait()
    pltpu.async_copy(rows_v, xs_hbm.at[d2_v], sem).wait()


def _dispatch(flat_tokens, d1, d2):
    mesh = plsc.VectorSubcoreMesh(core_axis_name="c", subcore_axis_name="s")
    return pl.kernel(
        _dispatch_body,
        out_type=jax.ShapeDtypeStruct((XS_TOT, H), jnp.float32),
        mesh=mesh,
        scratch_types=[
            pltpu.VMEM((TOK_PER,), jnp.int32),
            pltpu.VMEM((TOK_PER,), jnp.int32),
            pltpu.VMEM((TOK_PER, H), jnp.float32),
            pltpu.SemaphoreType.DMA,
        ],
    )(flat_tokens, d1, d2)


def _expert_body(be_ref, xs_ref, w1_ref, b1_ref, w2_ref, b2_ref, ys_ref):
    g = pl.program_id(0)

    @pl.when(be_ref[g] < E)
    def _():
        x = xs_ref[...]
        h = lax.dot_general(x, w1_ref[0], (((1,), (1,)), ((), ())),
                            preferred_element_type=jnp.float32)
        h = h + b1_ref[0]
        h = h * (1.0 / (1.0 + jnp.exp(-h)))
        y = lax.dot_general(h, w2_ref[0], (((1,), (1,)), ((), ())),
                            preferred_element_type=jnp.float32)
        ys_ref[...] = y + b2_ref[0]


def _experts(be, xs, w1, b1, w2, b2):
    def emap(g, be_s):
        return (jnp.minimum(be_s[g], E - 1), 0, 0)

    grid_spec = pltpu.PrefetchScalarGridSpec(
        num_scalar_prefetch=1,
        grid=(GMAX,),
        in_specs=[
            pl.BlockSpec((T, H), lambda g, be_s: (g, 0)),
            pl.BlockSpec((1, H, H), emap),
            pl.BlockSpec((1, 1, H), emap),
            pl.BlockSpec((1, H, H), emap),
            pl.BlockSpec((1, 1, H), emap),
        ],
        out_specs=pl.BlockSpec((T, H), lambda g, be_s: (g, 0)),
    )
    return pl.pallas_call(
        _expert_body,
        grid_spec=grid_spec,
        out_shape=jax.ShapeDtypeStruct((XS_ROWS, H), jnp.float32),
        compiler_params=pltpu.CompilerParams(
            dimension_semantics=("arbitrary",)),
    )(be, xs, w1, b1.reshape(E, 1, H), w2, b2.reshape(E, 1, H))


def _combine_body(ys_hbm, d1_hbm, d2_hbm, wa_hbm, wb_hbm, out_hbm,
                  d1_v, d2_v, wa_v, wb_v, r1_v, r2_v, o_v, sem):
    c = lax.axis_index("c")
    s = lax.axis_index("s")
    wid = s * NC + c
    sub = TOK_PER // 2
    for half in range(2):
        base = wid * TOK_PER + half * sub
        pltpu.sync_copy(d1_hbm.at[pl.ds(base, sub)], d1_v)
        pltpu.sync_copy(d2_hbm.at[pl.ds(base, sub)], d2_v)
        pltpu.sync_copy(wa_hbm.at[pl.ds(base, sub)], wa_v)
        pltpu.sync_copy(wb_hbm.at[pl.ds(base, sub)], wb_v)
        for k in range(sub // LANES):
            sl = pl.ds(k * LANES, LANES)
            d1_v[sl] = jnp.minimum(d1_v[sl], XS_ROWS - 1)
            d2_v[sl] = jnp.minimum(d2_v[sl], XS_ROWS - 1)
        pltpu.async_copy(ys_hbm.at[d1_v], r1_v, sem).wait()
        pltpu.async_copy(ys_hbm.at[d2_v], r2_v, sem).wait()

        def row(j, _):
            wa = wa_v[j, :]
            wb = wb_v[j, :]
            zero = jnp.zeros((LANES,), jnp.float32)
            for ch in range(H // LANES):
                sl = pl.ds(ch * LANES, LANES)
                a = r1_v[j, sl]
                b = r2_v[j, sl]
                o_v[j, sl] = (jnp.where(wa == 0.0, zero, a * wa)
                              + jnp.where(wb == 0.0, zero, b * wb))
            return 0

        lax.fori_loop(0, sub, row, 0)
        pltpu.sync_copy(o_v, out_hbm.at[pl.ds(base, sub)])


def _combine(ys, d1, d2, wa, wb):
    mesh = plsc.VectorSubcoreMesh(core_axis_name="c", subcore_axis_name="s")
    sub = TOK_PER // 2
    return pl.kernel(
        _combine_body,
        out_type=jax.ShapeDtypeStruct((N, H), jnp.float32),
        mesh=mesh,
        scratch_types=[
            pltpu.VMEM((sub,), jnp.int32),
            pltpu.VMEM((sub,), jnp.int32),
            pltpu.VMEM((sub, LANES), jnp.float32),
            pltpu.VMEM((sub, LANES), jnp.float32),
            pltpu.VMEM((sub, H), jnp.float32),
            pltpu.VMEM((sub, H), jnp.float32),
            pltpu.VMEM((sub, H), jnp.float32),
            pltpu.SemaphoreType.DMA,
        ],
    )(ys, d1, d2, wa, wb)


def kernel(tokens, W_gate, W1, b1, W2, b2):
    batch, seq, hidden = tokens.shape
    flat = tokens.reshape(batch * seq, hidden)
    d1, d2, wa, wb, be = _routing(flat, W_gate)
    d1 = d1.reshape(N)
    d2 = d2.reshape(N)
    xs = _dispatch(flat, d1, d2)
    ys = _experts(be.reshape(GPAD), xs, W1, b1, W2, b2)
    out = _combine(ys, d1, d2, wa, wb)
    return out.reshape(batch, seq, hidden)


# speedup vs baseline: 1.1136x; 1.0430x over previous
import functools

import jax
import jax.numpy as jnp
from jax import lax
from jax.experimental import pallas as pl
from jax.experimental.pallas import tpu as pltpu
from jax.experimental.pallas import tpu_sc as plsc

N = 2048
H = 768
E = 8
CAP = 1024
T = 256
GMAX = 23
GPAD = 32
XS_ROWS = GMAX * T
TRASH = XS_ROWS
XS_TOT = XS_ROWS + 8

NC, NS = 2, 16
NW = NC * NS
TOK_PER = N // NW
LANES = 16


def _routing_body(tok_ref, wg_ref, d1_ref, d2_ref, wa_ref, wb_ref, be_ref):
    x = tok_ref[...]
    wg = wg_ref[...]
    logits = lax.dot_general(x, wg, (((1,), (1,)), ((), ())),
                             preferred_element_type=jnp.float32)
    m = jnp.max(logits, axis=1, keepdims=True)
    ex = jnp.exp(logits - m)
    probs = ex / jnp.sum(ex, axis=1, keepdims=True)

    lane = lax.broadcasted_iota(jnp.int32, (N, E), 1)
    m1 = jnp.max(probs, axis=1, keepdims=True)
    i1 = jnp.min(jnp.where(probs == m1, lane, E), axis=1, keepdims=True)
    probs2 = jnp.where(lane == i1, -1.0, probs)
    m2 = jnp.max(probs2, axis=1, keepdims=True)
    i2 = jnp.min(jnp.where(probs2 == m2, lane, E), axis=1, keepdims=True)

    oh1 = (lane == i1).astype(jnp.float32)
    oh2 = (lane == i2).astype(jnp.float32)
    s = oh1 + oh2

    kc = 256
    row_i = lax.broadcasted_iota(jnp.int32, (N, kc), 0)
    col_i = lax.broadcasted_iota(jnp.int32, (N, kc), 1)
    c = jnp.zeros((N, E), jnp.float32)
    for k in range(N // kc):
        mask = (col_i + k * kc < row_i).astype(jnp.float32)
        c = c + lax.dot_general(mask, s[k * kc:(k + 1) * kc, :],
                                (((1,), (0,)), ((), ())),
                                preferred_element_type=jnp.float32)

    counts = jnp.sum(s, axis=0, keepdims=True)
    counts_i = counts.astype(jnp.int32)
    kept = counts_i <= CAP
    nblk = jnp.where(kept, (counts_i + (T - 1)) // T, 0)

    e_r = lax.broadcasted_iota(jnp.int32, (E, E), 0)
    e_c = lax.broadcasted_iota(jnp.int32, (E, E), 1)
    mex = (e_r < e_c).astype(jnp.float32)
    nblk_f = nblk.astype(jnp.float32)
    blkoff = lax.dot_general(nblk_f, mex, (((1,), (0,)), ((), ())),
                             preferred_element_type=jnp.float32)
    rowoff = blkoff * float(T)
    cum = blkoff + nblk_f

    r1 = jnp.sum(c * oh1, axis=1, keepdims=True)
    r2 = jnp.sum(c * oh2, axis=1, keepdims=True)
    ro1 = jnp.sum(rowoff * oh1, axis=1, keepdims=True)
    ro2 = jnp.sum(rowoff * oh2, axis=1, keepdims=True)
    keptf = kept.astype(jnp.float32)
    k1 = jnp.sum(keptf * oh1, axis=1, keepdims=True)
    k2 = jnp.sum(keptf * oh2, axis=1, keepdims=True)

    d1_ref[...] = jnp.where(k1 > 0., ro1 + r1, float(TRASH)).astype(jnp.int32)
    d2_ref[...] = jnp.where(k2 > 0., ro2 + r2, float(TRASH)).astype(jnp.int32)
    wa_ref[...] = jnp.broadcast_to(m1 * k1, (N, LANES))
    wb_ref[...] = jnp.broadcast_to(m2 * k2, (N, LANES))

    g_i = lax.broadcasted_iota(jnp.int32, (GPAD, E), 0).astype(jnp.float32)
    be = jnp.sum((cum <= g_i).astype(jnp.float32), axis=1, keepdims=True)
    be_ref[...] = be.astype(jnp.int32)


def _routing(flat_tokens, w_gate):
    return pl.pallas_call(
        _routing_body,
        out_shape=[
            jax.ShapeDtypeStruct((N, 1), jnp.int32),
            jax.ShapeDtypeStruct((N, 1), jnp.int32),
            jax.ShapeDtypeStruct((N, LANES), jnp.float32),
            jax.ShapeDtypeStruct((N, LANES), jnp.float32),
            jax.ShapeDtypeStruct((GPAD, 1), jnp.int32),
        ],
    )(flat_tokens, w_gate)


def _dispatch_body(tok_hbm, d1_hbm, d2_hbm, xs_hbm, d1_v, d2_v, rows_v,
                   sem_t, sem1, sem2):
    c = lax.axis_index("c")
    s = lax.axis_index("s")
    wid = s * NC + c
    base = wid * TOK_PER
    tok_cp = pltpu.async_copy(tok_hbm.at[pl.ds(base, TOK_PER)], rows_v, sem_t)
    pltpu.sync_copy(d1_hbm.at[pl.ds(base, TOK_PER)], d1_v)
    pltpu.sync_copy(d2_hbm.at[pl.ds(base, TOK_PER)], d2_v)
    tok_cp.wait()
    cp1 = pltpu.async_copy(rows_v, xs_hbm.at[d1_v], sem1)
    cp2 = pltpu.async_copy(rows_v, xs_hbm.at[d2_v], sem2)
    cp1.wait()
    cp2.wait()


def _dispatch(flat_tokens, d1, d2):
    mesh = plsc.VectorSubcoreMesh(core_axis_name="c", subcore_axis_name="s")
    return pl.kernel(
        _dispatch_body,
        out_type=jax.ShapeDtypeStruct((XS_TOT, H), jnp.float32),
        mesh=mesh,
        scratch_types=[
            pltpu.VMEM((TOK_PER,), jnp.int32),
            pltpu.VMEM((TOK_PER,), jnp.int32),
            pltpu.VMEM((TOK_PER, H), jnp.float32),
            pltpu.SemaphoreType.DMA,
            pltpu.SemaphoreType.DMA,
            pltpu.SemaphoreType.DMA,
        ],
    )(flat_tokens, d1, d2)


def _expert_body(be_ref, xs_ref, w1_ref, b1_ref, w2_ref, b2_ref, ys_ref):
    g = pl.program_id(0)

    @pl.when(be_ref[g] < E)
    def _():
        x = xs_ref[...].astype(jnp.bfloat16)
        h = lax.dot_general(x, w1_ref[0].astype(jnp.bfloat16),
                            (((1,), (1,)), ((), ())),
                            preferred_element_type=jnp.float32)
        h = h + b1_ref[0]
        h = h * (1.0 / (1.0 + jnp.exp(-h)))
        y = lax.dot_general(h.astype(jnp.bfloat16),
                            w2_ref[0].astype(jnp.bfloat16),
                            (((1,), (1,)), ((), ())),
                            preferred_element_type=jnp.float32)
        ys_ref[...] = y + b2_ref[0]


def _experts(be, xs, w1, b1, w2, b2):
    def emap(g, be_s):
        return (jnp.minimum(be_s[g], E - 1), 0, 0)

    grid_spec = pltpu.PrefetchScalarGridSpec(
        num_scalar_prefetch=1,
        grid=(GMAX,),
        in_specs=[
            pl.BlockSpec((T, H), lambda g, be_s: (g, 0)),
            pl.BlockSpec((1, H, H), emap),
            pl.BlockSpec((1, 1, H), emap),
            pl.BlockSpec((1, H, H), emap),
            pl.BlockSpec((1, 1, H), emap),
        ],
        out_specs=pl.BlockSpec((T, H), lambda g, be_s: (g, 0)),
    )
    return pl.pallas_call(
        _expert_body,
        grid_spec=grid_spec,
        out_shape=jax.ShapeDtypeStruct((XS_ROWS, H), jnp.float32),
        compiler_params=pltpu.CompilerParams(
            dimension_semantics=("arbitrary",)),
    )(be, xs, w1, b1.reshape(E, 1, H), w2, b2.reshape(E, 1, H))


def _combine_body(ys_hbm, d1_hbm, d2_hbm, wa_hbm, wb_hbm, out_hbm,
                  d1_v, d2_v, wa_v, wb_v, r1_v, r2_v, sem1, sem2):
    c = lax.axis_index("c")
    s = lax.axis_index("s")
    wid = s * NC + c
    base = wid * TOK_PER
    pltpu.sync_copy(d1_hbm.at[pl.ds(base, TOK_PER)], d1_v)
    pltpu.sync_copy(d2_hbm.at[pl.ds(base, TOK_PER)], d2_v)
    pltpu.sync_copy(wa_hbm.at[pl.ds(base, TOK_PER)], wa_v)
    pltpu.sync_copy(wb_hbm.at[pl.ds(base, TOK_PER)], wb_v)
    for k in range(TOK_PER // LANES):
        sl = pl.ds(k * LANES, LANES)
        d1_v[sl] = jnp.minimum(d1_v[sl], XS_ROWS - 1)
        d2_v[sl] = jnp.minimum(d2_v[sl], XS_ROWS - 1)
    cp1 = pltpu.async_copy(ys_hbm.at[d1_v], r1_v, sem1)
    cp2 = pltpu.async_copy(ys_hbm.at[d2_v], r2_v, sem2)
    cp1.wait()
    cp2.wait()

    def row(j, _):
        wa = wa_v[j, :]
        wb = wb_v[j, :]
        zero = jnp.zeros((LANES,), jnp.float32)
        for ch in range(H // LANES):
            sl = pl.ds(ch * LANES, LANES)
            a = r1_v[j, sl]
            b = r2_v[j, sl]
            r1_v[j, sl] = (jnp.where(wa == 0.0, zero, a * wa)
                           + jnp.where(wb == 0.0, zero, b * wb))
        return 0

    lax.fori_loop(0, TOK_PER, row, 0)
    pltpu.sync_copy(r1_v, out_hbm.at[pl.ds(base, TOK_PER)])


def _combine(ys, d1, d2, wa, wb):
    mesh = plsc.VectorSubcoreMesh(core_axis_name="c", subcore_axis_name="s")
    return pl.kernel(
        _combine_body,
        out_type=jax.ShapeDtypeStruct((N, H), jnp.float32),
        mesh=mesh,
        scratch_types=[
            pltpu.VMEM((TOK_PER,), jnp.int32),
            pltpu.VMEM((TOK_PER,), jnp.int32),
            pltpu.VMEM((TOK_PER, LANES), jnp.float32),
            pltpu.VMEM((TOK_PER, LANES), jnp.float32),
            pltpu.VMEM((TOK_PER, H), jnp.float32),
            pltpu.VMEM((TOK_PER, H), jnp.float32),
            pltpu.SemaphoreType.DMA,
            pltpu.SemaphoreType.DMA,
        ],
    )(ys, d1, d2, wa, wb)


def kernel(tokens, W_gate, W1, b1, W2, b2):
    batch, seq, hidden = tokens.shape
    flat = tokens.reshape(batch * seq, hidden)
    d1, d2, wa, wb, be = _routing(flat, W_gate)
    d1 = d1.reshape(N)
    d2 = d2.reshape(N)
    xs = _dispatch(flat, d1, d2)
    ys = _experts(be.reshape(GPAD), xs, W1, b1, W2, b2)
    out = _combine(ys, d1, d2, wa, wb)
    return out.reshape(batch, seq, hidden)


# speedup vs baseline: 1.2430x; 1.1162x over previous
import functools

import jax
import jax.numpy as jnp
from jax import lax
from jax.experimental import pallas as pl
from jax.experimental.pallas import tpu as pltpu
from jax.experimental.pallas import tpu_sc as plsc

N = 2048
H = 768
E = 8
CAP = 1024
XS_ROWS = E * CAP
TRASH = XS_ROWS
XS_TOT = XS_ROWS + 8

NC, NS = 2, 16
NW = NC * NS
TOK_PER = N // NW
LANES = 16


def _routing_body(tok_ref, wg_ref, d1_ref, d2_ref, wa_ref, wb_ref, cnt_ref):
    x = tok_ref[...]
    wg = wg_ref[...]
    logits = lax.dot_general(x, wg, (((1,), (1,)), ((), ())),
                             preferred_element_type=jnp.float32)
    m = jnp.max(logits, axis=1, keepdims=True)
    ex = jnp.exp(logits - m)
    probs = ex / jnp.sum(ex, axis=1, keepdims=True)

    lane = lax.broadcasted_iota(jnp.int32, (N, E), 1)
    m1 = jnp.max(probs, axis=1, keepdims=True)
    i1 = jnp.min(jnp.where(probs == m1, lane, E), axis=1, keepdims=True)
    probs2 = jnp.where(lane == i1, -1.0, probs)
    m2 = jnp.max(probs2, axis=1, keepdims=True)
    i2 = jnp.min(jnp.where(probs2 == m2, lane, E), axis=1, keepdims=True)

    oh1 = (lane == i1).astype(jnp.float32)
    oh2 = (lane == i2).astype(jnp.float32)
    s = oh1 + oh2

    kc = 256
    row_i = lax.broadcasted_iota(jnp.int32, (N, kc), 0)
    col_i = lax.broadcasted_iota(jnp.int32, (N, kc), 1)
    c = jnp.zeros((N, E), jnp.float32)
    for k in range(N // kc):
        mask = (col_i + k * kc < row_i).astype(jnp.float32)
        c = c + lax.dot_general(mask, s[k * kc:(k + 1) * kc, :],
                                (((1,), (0,)), ((), ())),
                                preferred_element_type=jnp.float32)

    counts = jnp.sum(s, axis=0, keepdims=True)
    counts_i = counts.astype(jnp.int32)
    kept = counts_i <= CAP

    r1 = jnp.sum(c * oh1, axis=1, keepdims=True)
    r2 = jnp.sum(c * oh2, axis=1, keepdims=True)
    keptf = kept.astype(jnp.float32)
    k1 = jnp.sum(keptf * oh1, axis=1, keepdims=True)
    k2 = jnp.sum(keptf * oh2, axis=1, keepdims=True)
    ro1 = i1.astype(jnp.float32) * float(CAP)
    ro2 = i2.astype(jnp.float32) * float(CAP)

    d1_ref[...] = jnp.where(k1 > 0., ro1 + r1, float(TRASH)).astype(jnp.int32)
    d2_ref[...] = jnp.where(k2 > 0., ro2 + r2, float(TRASH)).astype(jnp.int32)
    wa_ref[...] = jnp.broadcast_to(m1 * k1, (N, LANES))
    wb_ref[...] = jnp.broadcast_to(m2 * k2, (N, LANES))
    cnt_ref[...] = counts_i


def _routing(flat_tokens, w_gate):
    return pl.pallas_call(
        _routing_body,
        out_shape=[
            jax.ShapeDtypeStruct((N, 1), jnp.int32),
            jax.ShapeDtypeStruct((N, 1), jnp.int32),
            jax.ShapeDtypeStruct((N, LANES), jnp.float32),
            jax.ShapeDtypeStruct((N, LANES), jnp.float32),
            jax.ShapeDtypeStruct((1, E), jnp.int32),
        ],
    )(flat_tokens, w_gate)


def _dispatch_body(tok_hbm, d1_hbm, d2_hbm, xs_hbm, d1_v, d2_v, rows_v,
                   sem_t, sem1, sem2):
    c = lax.axis_index("c")
    s = lax.axis_index("s")
    wid = s * NC + c
    base = wid * TOK_PER
    tok_cp = pltpu.async_copy(tok_hbm.at[pl.ds(base, TOK_PER)], rows_v, sem_t)
    pltpu.sync_copy(d1_hbm.at[pl.ds(base, TOK_PER)], d1_v)
    pltpu.sync_copy(d2_hbm.at[pl.ds(base, TOK_PER)], d2_v)
    tok_cp.wait()
    cp1 = pltpu.async_copy(rows_v, xs_hbm.at[d1_v], sem1)
    cp2 = pltpu.async_copy(rows_v, xs_hbm.at[d2_v], sem2)
    cp1.wait()
    cp2.wait()


def _dispatch(flat_tokens, d1, d2):
    mesh = plsc.VectorSubcoreMesh(core_axis_name="c", subcore_axis_name="s")
    return pl.kernel(
        _dispatch_body,
        out_type=jax.ShapeDtypeStruct((XS_TOT, H), jnp.float32),
        mesh=mesh,
        scratch_types=[
            pltpu.VMEM((TOK_PER,), jnp.int32),
            pltpu.VMEM((TOK_PER,), jnp.int32),
            pltpu.VMEM((TOK_PER, H), jnp.float32),
            pltpu.SemaphoreType.DMA,
            pltpu.SemaphoreType.DMA,
            pltpu.SemaphoreType.DMA,
        ],
    )(flat_tokens, d1, d2)


def _expert_body(cnt_ref, xs_ref, w1_ref, b1_ref, w2_ref, b2_ref, ys_ref):
    e = pl.program_id(0)
    cnt = cnt_ref[e]

    @pl.when((cnt > 0) & (cnt <= CAP))
    def _():
        x = xs_ref[...].astype(jnp.bfloat16)
        h = lax.dot_general(x, w1_ref[0].astype(jnp.bfloat16),
                            (((1,), (1,)), ((), ())),
                            preferred_element_type=jnp.float32)
        h = h + b1_ref[0]
        h = h * (1.0 / (1.0 + jnp.exp(-h)))
        y = lax.dot_general(h.astype(jnp.bfloat16),
                            w2_ref[0].astype(jnp.bfloat16),
                            (((1,), (1,)), ((), ())),
                            preferred_element_type=jnp.float32)
        ys_ref[...] = y + b2_ref[0]


def _experts(cnt, xs, w1, b1, w2, b2):
    def emap(e, cnt_s):
        return (e, 0, 0)

    grid_spec = pltpu.PrefetchScalarGridSpec(
        num_scalar_prefetch=1,
        grid=(E,),
        in_specs=[
            pl.BlockSpec((CAP, H), lambda e, cnt_s: (e, 0)),
            pl.BlockSpec((1, H, H), emap),
            pl.BlockSpec((1, 1, H), emap),
            pl.BlockSpec((1, H, H), emap),
            pl.BlockSpec((1, 1, H), emap),
        ],
        out_specs=pl.BlockSpec((CAP, H), lambda e, cnt_s: (e, 0)),
    )
    return pl.pallas_call(
        _expert_body,
        grid_spec=grid_spec,
        out_shape=jax.ShapeDtypeStruct((XS_ROWS, H), jnp.float32),
        compiler_params=pltpu.CompilerParams(
            dimension_semantics=("arbitrary",)),
    )(cnt, xs, w1, b1.reshape(E, 1, H), w2, b2.reshape(E, 1, H))


def _combine_body(ys_hbm, d1_hbm, d2_hbm, wa_hbm, wb_hbm, out_hbm,
                  d1_v, d2_v, wa_v, wb_v, r1_v, r2_v, sem1, sem2):
    c = lax.axis_index("c")
    s = lax.axis_index("s")
    wid = s * NC + c
    base = wid * TOK_PER
    pltpu.sync_copy(d1_hbm.at[pl.ds(base, TOK_PER)], d1_v)
    pltpu.sync_copy(d2_hbm.at[pl.ds(base, TOK_PER)], d2_v)
    pltpu.sync_copy(wa_hbm.at[pl.ds(base, TOK_PER)], wa_v)
    pltpu.sync_copy(wb_hbm.at[pl.ds(base, TOK_PER)], wb_v)
    for k in range(TOK_PER // LANES):
        sl = pl.ds(k * LANES, LANES)
        d1_v[sl] = jnp.minimum(d1_v[sl], XS_ROWS - 1)
        d2_v[sl] = jnp.minimum(d2_v[sl], XS_ROWS - 1)
    cp1 = pltpu.async_copy(ys_hbm.at[d1_v], r1_v, sem1)
    cp2 = pltpu.async_copy(ys_hbm.at[d2_v], r2_v, sem2)
    cp1.wait()
    cp2.wait()

    def row(j, _):
        wa = wa_v[j, :]
        wb = wb_v[j, :]
        zero = jnp.zeros((LANES,), jnp.float32)
        for ch in range(H // LANES):
            sl = pl.ds(ch * LANES, LANES)
            a = r1_v[j, sl]
            b = r2_v[j, sl]
            r1_v[j, sl] = (jnp.where(wa == 0.0, zero, a * wa)
                           + jnp.where(wb == 0.0, zero, b * wb))
        return 0

    lax.fori_loop(0, TOK_PER, row, 0)
    pltpu.sync_copy(r1_v, out_hbm.at[pl.ds(base, TOK_PER)])


def _combine(ys, d1, d2, wa, wb):
    mesh = plsc.VectorSubcoreMesh(core_axis_name="c", subcore_axis_name="s")
    return pl.kernel(
        _combine_body,
        out_type=jax.ShapeDtypeStruct((N, H), jnp.float32),
        mesh=mesh,
        scratch_types=[
            pltpu.VMEM((TOK_PER,), jnp.int32),
            pltpu.VMEM((TOK_PER,), jnp.int32),
            pltpu.VMEM((TOK_PER, LANES), jnp.float32),
            pltpu.VMEM((TOK_PER, LANES), jnp.float32),
            pltpu.VMEM((TOK_PER, H), jnp.float32),
            pltpu.VMEM((TOK_PER, H), jnp.float32),
            pltpu.SemaphoreType.DMA,
            pltpu.SemaphoreType.DMA,
        ],
    )(ys, d1, d2, wa, wb)


def kernel(tokens, W_gate, W1, b1, W2, b2):
    batch, seq, hidden = tokens.shape
    flat = tokens.reshape(batch * seq, hidden)
    d1, d2, wa, wb, cnt = _routing(flat, W_gate)
    d1 = d1.reshape(N)
    d2 = d2.reshape(N)
    xs = _dispatch(flat, d1, d2)
    ys = _experts(cnt.reshape(E), xs, W1, b1, W2, b2)
    out = _combine(ys, d1, d2, wa, wb)
    return out.reshape(batch, seq, hidden)
